# baseline (speedup 1.0000x reference)
import functools

import jax
import jax.numpy as jnp
from jax import lax
from jax.experimental import pallas as pl
from jax.experimental.pallas import tpu as pltpu
from jax.experimental.pallas import tpu_sc as plsc

_VOCAB = 128
_D = 32
_ROWS = 4096
_COLS = 200
_B = _ROWS * _COLS
_NC = 2
_NS = 16
_NW = _NC * _NS
_BPW = _B // _NW
_IDXW = 128
_K = 1024
_NSUB = _K // _IDXW
_NCH = _BPW // _K
_IDX_ROWS_PER_W = _BPW // _IDXW


def _emb_body(ids_hbm, table_hbm, out_hbm, idx_v, rows_v, sem_g, sem_o):
    wid = lax.axis_index("s") * _NC + lax.axis_index("c")
    idx_base = wid * _IDX_ROWS_PER_W
    out_base = wid * _BPW

    def stage_idx(i, slot):
        pltpu.sync_copy(
            ids_hbm.at[pl.ds(idx_base + i * _NSUB, _NSUB)],
            idx_v.at[pl.ds(slot * _NSUB, _NSUB)],
        )

    def fire(slot):
        def one(j, c):
            pltpu.async_copy(
                table_hbm.at[idx_v.at[slot * _NSUB + j]],
                rows_v.at[pl.ds(slot * _K + j * _IDXW, _IDXW)],
                sem_g,
            )
            return c
        lax.fori_loop(0, _NSUB, one, 0)

    def drain_gathers():
        pltpu.make_async_copy(
            out_hbm.at[pl.ds(out_base, _K)],
            rows_v.at[pl.ds(0, _K)],
            sem_g,
        ).wait()

    def writeout(i, slot):
        pltpu.async_copy(
            rows_v.at[pl.ds(slot * _K, _K)],
            out_hbm.at[pl.ds(out_base + i * _K, _K)],
            sem_o,
        )

    def drain_writeout():
        pltpu.make_async_copy(
            rows_v.at[pl.ds(0, _K)],
            out_hbm.at[pl.ds(out_base, _K)],
            sem_o,
        ).wait()

    stage_idx(0, 0)
    fire(0)
    stage_idx(1, 1)
    drain_gathers()
    writeout(0, 0)
    fire(1)

    def body(i, c):
        nslot = (i + 1) % 2
        stage_idx(i + 1, nslot)
        drain_gathers()
        writeout(i, i % 2)
        drain_writeout()
        fire(nslot)
        return c

    lax.fori_loop(1, _NCH - 1, body, 0)

    drain_gathers()
    writeout(_NCH - 1, (_NCH - 1) % 2)
    drain_writeout()
    drain_writeout()


_emb = functools.partial(
    pl.kernel,
    mesh=plsc.VectorSubcoreMesh(core_axis_name="c", subcore_axis_name="s"),
    out_type=jax.ShapeDtypeStruct((_B, _D), jnp.float32),
    scratch_types=[
        pltpu.VMEM((2 * _NSUB, _IDXW), jnp.int32),
        pltpu.VMEM((2 * _K, _D), jnp.float32),
        pltpu.SemaphoreType.DMA,
        pltpu.SemaphoreType.DMA,
    ],
    compiler_params=pltpu.CompilerParams(use_tc_tiling_on_sc=False),
)(_emb_body)


@jax.jit
def kernel(input_ids, embed_weight):
    ids = input_ids.astype(jnp.int32).reshape(_B // _IDXW, _IDXW)
    out = _emb(ids, embed_weight)
    return out.reshape(_ROWS, _COLS, _D)
